# baseline (speedup 1.0000x reference)
import jax
import jax.numpy as jnp
from jax import lax
from jax.experimental import pallas as pl
from jax.experimental.pallas import tpu as pltpu
from jax.experimental.pallas import tpu_sc as plsc

_B = 16384
_DIM = 32
_K = 3
_DW = _DIM * (2 * _K + 1)
_NM = 100000
_ND = 100000
_NC, _NS, _L = 2, 16, 16
_NW = _NC * _NS
_RPW = _B // _NW
_CH = 128
_NCH = _RPW // _CH
_PR = 2000


def _prep_body(d_ref, dp_ref):
    dp_ref[:, 0:_DW] = d_ref[...]
    dp_ref[:, _DW:256] = jnp.zeros((_PR, 256 - _DW), jnp.float32)


def _sc_body(ij0, ij1, ipx, m_bar, d_bar, m_pack, d_pack, out,
             idx0_v, idx1_v, idxp_v, i0d_v, ipd_v, mi_v, mp_v, dd_v,
             mb_v, db_v, o_v, sem):
    wid = lax.axis_index("s") * _NC + lax.axis_index("c")
    base = wid * _RPW
    pltpu.sync_copy(ij0.at[pl.ds(base, _RPW)], idx0_v)
    pltpu.sync_copy(ij1.at[pl.ds(base, _RPW)], idx1_v)
    pltpu.sync_copy(ipx.at[pl.ds(base, _RPW)], idxp_v)
    iota = lax.broadcasted_iota(jnp.int32, (_L,), 0)

    def divstep(g, _):
        v0 = idx0_v[pl.ds(g * _L, _L)]
        i0d_v[pl.ds(g * _L, _L)] = lax.shift_right_logical(v0, 2)
        vp = idxp_v[pl.ds(g * _L, _L)]
        ipd_v[pl.ds(g * _L, _L)] = lax.shift_right_logical(vp, 2)
        return 0

    lax.fori_loop(0, _RPW // _L, divstep, 0)

    for c in range(_NCH):
        i0 = idx0_v.at[pl.ds(c * _CH, _CH)]
        i1 = idx1_v.at[pl.ds(c * _CH, _CH)]
        i0d = i0d_v.at[pl.ds(c * _CH, _CH)]
        ipd = ipd_v.at[pl.ds(c * _CH, _CH)]
        cps = [
            pltpu.async_copy(d_pack.at[i1], dd_v, sem),
            pltpu.async_copy(m_pack.at[i0d], mi_v, sem),
            pltpu.async_copy(m_pack.at[ipd], mp_v, sem),
            pltpu.async_copy(m_bar.at[i0], mb_v, sem),
            pltpu.async_copy(d_bar.at[i1], db_v, sem),
        ]
        for cp in cps:
            cp.wait()

        def group(g, _):
            rows = g * _L + iota
            i0g = idx0_v[pl.ds(c * _CH + g * _L, _L)]
            ipg = idxp_v[pl.ds(c * _CH + g * _L, _L)]
            mi_c = lax.shift_left(jnp.bitwise_and(i0g, 3), 5)
            mp_c = lax.shift_left(jnp.bitwise_and(ipg, 3), 5)
            acc0 = mb_v[pl.ds(g * _L, _L)] + db_v[pl.ds(g * _L, _L)]
            zero = jnp.zeros((_L,), jnp.float32)

            def dstep(d, carry):
                acc, s0, s1, s2, t0, t1, t2 = carry
                mi = plsc.load_gather(mi_v, [rows, mi_c + d])
                mp = plsc.load_gather(mp_v, [rows, mp_c + d])
                dj = plsc.load_gather(dd_v, [rows, jnp.full((_L,), d, jnp.int32)])
                acc = acc + mi * dj
                cs = jnp.full((_L,), _DIM + d * _K, jnp.int32)
                s0 = s0 + mi * plsc.load_gather(dd_v, [rows, cs])
                s1 = s1 + mi * plsc.load_gather(dd_v, [rows, cs + 1])
                s2 = s2 + mi * plsc.load_gather(dd_v, [rows, cs + 2])
                cg = cs + _K * _DIM
                t0 = t0 + mp * plsc.load_gather(dd_v, [rows, cg])
                t1 = t1 + mp * plsc.load_gather(dd_v, [rows, cg + 1])
                t2 = t2 + mp * plsc.load_gather(dd_v, [rows, cg + 2])
                return acc, s0, s1, s2, t0, t1, t2

            acc, s0, s1, s2, t0, t1, t2 = lax.fori_loop(
                0, _DIM, dstep, (acc0, zero, zero, zero, zero, zero, zero),
                unroll=4)
            o_v[pl.ds(g * _L, _L)] = acc + s0 * t0 + s1 * t1 + s2 * t2
            return 0

        lax.fori_loop(0, _CH // _L, group, 0)
        pltpu.sync_copy(o_v, out.at[pl.ds(base + c * _CH, _CH)])


@jax.jit
def _run(ij0, ij1, ipx, m_bar, d_bar, m_pack, d_tab):
    d_pack = pl.pallas_call(
        _prep_body,
        grid=(_ND // _PR,),
        in_specs=[pl.BlockSpec((_PR, _DW), lambda i: (i, 0))],
        out_specs=pl.BlockSpec((_PR, 256), lambda i: (i, 0)),
        out_shape=jax.ShapeDtypeStruct((_ND, 256), jnp.float32),
    )(d_tab)

    mesh = plsc.VectorSubcoreMesh(core_axis_name="c", subcore_axis_name="s")
    f = pl.kernel(
        _sc_body,
        out_type=jax.ShapeDtypeStruct((_B,), jnp.float32),
        mesh=mesh,
        scratch_types=[
            pltpu.VMEM((_RPW,), jnp.int32),
            pltpu.VMEM((_RPW,), jnp.int32),
            pltpu.VMEM((_RPW,), jnp.int32),
            pltpu.VMEM((_RPW,), jnp.int32),
            pltpu.VMEM((_RPW,), jnp.int32),
            pltpu.VMEM((_CH, 128), jnp.float32),
            pltpu.VMEM((_CH, 128), jnp.float32),
            pltpu.VMEM((_CH, 256), jnp.float32),
            pltpu.VMEM((_CH,), jnp.float32),
            pltpu.VMEM((_CH,), jnp.float32),
            pltpu.VMEM((_CH,), jnp.float32),
            pltpu.SemaphoreType.DMA,
        ],
        compiler_params=pltpu.CompilerParams(needs_layout_passes=False),
    )
    return f(ij0, ij1, ipx, m_bar, d_bar, m_pack, d_pack)


def kernel(ij, ip, m_bar, d_bar, M_table, D_table):
    ij0 = jnp.asarray(ij[:, 0], jnp.int32)
    ij1 = jnp.asarray(ij[:, 1], jnp.int32)
    m_pack = jnp.reshape(M_table, (M_table.shape[0] // 4, 4 * M_table.shape[1]))
    return _run(ij0, ij1, ip, m_bar, d_bar, m_pack, D_table)


# speedup vs baseline: 1.1688x; 1.1688x over previous
import jax
import jax.numpy as jnp
from jax import lax
from jax.experimental import pallas as pl
from jax.experimental.pallas import tpu as pltpu
from jax.experimental.pallas import tpu_sc as plsc

_B = 16384
_DIM = 32
_K = 3
_DW = _DIM * (2 * _K + 1)
_NM = 100000
_ND = 100000
_NC, _NS, _L = 2, 16, 16
_NW = _NC * _NS
_RPW = _B // _NW
_CH = 128
_NCH = _RPW // _CH
_PR = 2000


def _prep_body(m_ref, d_ref, mp_ref, dp_ref):
    mp_ref[...] = jnp.concatenate(
        [m_ref[...], jnp.zeros((_PR, 128 - _DIM), jnp.float32)], axis=1)
    dp_ref[...] = jnp.concatenate(
        [d_ref[...], jnp.zeros((_PR, 256 - _DW), jnp.float32)], axis=1)


def _sc_body(ij0, ij1, ipx, m_bar, d_bar, m_pad, d_pack, out,
             idx0_v, idx1_v, idxp_v, mi_v, mp_v, dd_v,
             mb_v, db_v, o_v, sem):
    wid = lax.axis_index("s") * _NC + lax.axis_index("c")
    base = wid * _RPW
    pltpu.sync_copy(ij0.at[pl.ds(base, _RPW)], idx0_v)
    pltpu.sync_copy(ij1.at[pl.ds(base, _RPW)], idx1_v)
    pltpu.sync_copy(ipx.at[pl.ds(base, _RPW)], idxp_v)
    iota = lax.broadcasted_iota(jnp.int32, (_L,), 0)

    for c in range(_NCH):
        i0 = idx0_v.at[pl.ds(c * _CH, _CH)]
        i1 = idx1_v.at[pl.ds(c * _CH, _CH)]
        ipc = idxp_v.at[pl.ds(c * _CH, _CH)]
        cps = [
            pltpu.async_copy(d_pack.at[i1], dd_v, sem),
            pltpu.async_copy(m_pad.at[i0], mi_v, sem),
            pltpu.async_copy(m_pad.at[ipc], mp_v, sem),
            pltpu.async_copy(m_bar.at[i0], mb_v, sem),
            pltpu.async_copy(d_bar.at[i1], db_v, sem),
        ]
        for cp in cps:
            cp.wait()

        def group(g, _):
            rows = g * _L + iota
            acc0 = mb_v[pl.ds(g * _L, _L)] + db_v[pl.ds(g * _L, _L)]
            zero = jnp.zeros((_L,), jnp.float32)

            def dstep(t, carry):
                acc, s0, s1, s2, t0, t1, t2 = carry
                dv = jnp.bitwise_and(t + iota, _DIM - 1)
                mi = plsc.load_gather(mi_v, [rows, dv])
                mp = plsc.load_gather(mp_v, [rows, dv])
                dj = plsc.load_gather(dd_v, [rows, dv])
                acc = acc + mi * dj
                cs = _DIM + dv + dv + dv
                s0 = s0 + mi * plsc.load_gather(dd_v, [rows, cs])
                s1 = s1 + mi * plsc.load_gather(dd_v, [rows, cs + 1])
                s2 = s2 + mi * plsc.load_gather(dd_v, [rows, cs + 2])
                cg = cs + _K * _DIM
                t0 = t0 + mp * plsc.load_gather(dd_v, [rows, cg])
                t1 = t1 + mp * plsc.load_gather(dd_v, [rows, cg + 1])
                t2 = t2 + mp * plsc.load_gather(dd_v, [rows, cg + 2])
                return acc, s0, s1, s2, t0, t1, t2

            acc, s0, s1, s2, t0, t1, t2 = lax.fori_loop(
                0, _DIM, dstep, (acc0, zero, zero, zero, zero, zero, zero),
                unroll=4)
            o_v[pl.ds(g * _L, _L)] = acc + s0 * t0 + s1 * t1 + s2 * t2
            return 0

        lax.fori_loop(0, _CH // _L, group, 0)
        pltpu.sync_copy(o_v, out.at[pl.ds(base + c * _CH, _CH)])


@jax.jit
def _run(ij0, ij1, ipx, m_bar, d_bar, m_tab, d_tab):
    m_pad, d_pack = pl.pallas_call(
        _prep_body,
        grid=(_ND // _PR,),
        in_specs=[
            pl.BlockSpec((_PR, _DIM), lambda i: (i, 0)),
            pl.BlockSpec((_PR, _DW), lambda i: (i, 0)),
        ],
        out_specs=[
            pl.BlockSpec((_PR, 128), lambda i: (i, 0)),
            pl.BlockSpec((_PR, 256), lambda i: (i, 0)),
        ],
        out_shape=[
            jax.ShapeDtypeStruct((_NM, 128), jnp.float32),
            jax.ShapeDtypeStruct((_ND, 256), jnp.float32),
        ],
    )(m_tab, d_tab)

    mesh = plsc.VectorSubcoreMesh(core_axis_name="c", subcore_axis_name="s")
    f = pl.kernel(
        _sc_body,
        out_type=jax.ShapeDtypeStruct((_B,), jnp.float32),
        mesh=mesh,
        scratch_types=[
            pltpu.VMEM((_RPW,), jnp.int32),
            pltpu.VMEM((_RPW,), jnp.int32),
            pltpu.VMEM((_RPW,), jnp.int32),
            pltpu.VMEM((_CH, 128), jnp.float32),
            pltpu.VMEM((_CH, 128), jnp.float32),
            pltpu.VMEM((_CH, 256), jnp.float32),
            pltpu.VMEM((_CH,), jnp.float32),
            pltpu.VMEM((_CH,), jnp.float32),
            pltpu.VMEM((_CH,), jnp.float32),
            pltpu.SemaphoreType.DMA,
        ],
        compiler_params=pltpu.CompilerParams(needs_layout_passes=False),
    )
    return f(ij0, ij1, ipx, m_bar, d_bar, m_pad, d_pack)


def kernel(ij, ip, m_bar, d_bar, M_table, D_table):
    ij0 = jnp.asarray(ij[:, 0], jnp.int32)
    ij1 = jnp.asarray(ij[:, 1], jnp.int32)
    return _run(ij0, ij1, ip, m_bar, d_bar, M_table, D_table)


# speedup vs baseline: 2.1924x; 1.8757x over previous
import jax
import jax.numpy as jnp
from jax import lax
from jax.experimental import pallas as pl
from jax.experimental.pallas import tpu as pltpu
from jax.experimental.pallas import tpu_sc as plsc

_B = 16384
_DIM = 32
_K = 3
_DW = _DIM * (2 * _K + 1)
_NM = 100000
_ND = 100000
_NC, _NS, _L = 2, 16, 16
_NW = _NC * _NS
_RPW = _B // _NW
_CH = 128
_NCH = _RPW // _CH
_PR = 2048


def _prep_body(mt_ref, dt_ref, mp_ref, dp_ref):
    m_t = jnp.transpose(mt_ref[...], (1, 0))
    mp_ref[...] = jnp.concatenate(
        [m_t, jnp.zeros((_PR, 128 - _DIM), jnp.float32)], axis=1)
    d_t = jnp.transpose(dt_ref[...], (1, 0))
    dp_ref[...] = jnp.concatenate(
        [d_t, jnp.zeros((_PR, 256 - _DW), jnp.float32)], axis=1)


def _sc_body(ij0, ij1, ipx, m_bar, d_bar, m_pad, d_pack, out,
             idx0_v, idx1_v, idxp_v, mi_v, mp_v, dd_v,
             mb_v, db_v, o_v, sem):
    wid = lax.axis_index("s") * _NC + lax.axis_index("c")
    base = wid * _RPW
    pltpu.sync_copy(ij0.at[pl.ds(base, _RPW)], idx0_v)
    pltpu.sync_copy(ij1.at[pl.ds(base, _RPW)], idx1_v)
    pltpu.sync_copy(ipx.at[pl.ds(base, _RPW)], idxp_v)
    iota = lax.broadcasted_iota(jnp.int32, (_L,), 0)

    for c in range(_NCH):
        i0 = idx0_v.at[pl.ds(c * _CH, _CH)]
        i1 = idx1_v.at[pl.ds(c * _CH, _CH)]
        ipc = idxp_v.at[pl.ds(c * _CH, _CH)]
        cps = [
            pltpu.async_copy(d_pack.at[i1], dd_v, sem),
            pltpu.async_copy(m_pad.at[i0], mi_v, sem),
            pltpu.async_copy(m_pad.at[ipc], mp_v, sem),
            pltpu.async_copy(m_bar.at[i0], mb_v, sem),
            pltpu.async_copy(d_bar.at[i1], db_v, sem),
        ]
        for cp in cps:
            cp.wait()

        def group(g, _):
            rows = g * _L + iota
            acc0 = mb_v[pl.ds(g * _L, _L)] + db_v[pl.ds(g * _L, _L)]
            zero = jnp.zeros((_L,), jnp.float32)

            def dstep(t, carry):
                acc, s0, s1, s2, t0, t1, t2 = carry
                dv = jnp.bitwise_and(t + iota, _DIM - 1)
                mi = plsc.load_gather(mi_v, [rows, dv])
                mp = plsc.load_gather(mp_v, [rows, dv])
                dj = plsc.load_gather(dd_v, [rows, dv])
                acc = acc + mi * dj
                cs = _DIM + dv + dv + dv
                s0 = s0 + mi * plsc.load_gather(dd_v, [rows, cs])
                s1 = s1 + mi * plsc.load_gather(dd_v, [rows, cs + 1])
                s2 = s2 + mi * plsc.load_gather(dd_v, [rows, cs + 2])
                cg = cs + _K * _DIM
                t0 = t0 + mp * plsc.load_gather(dd_v, [rows, cg])
                t1 = t1 + mp * plsc.load_gather(dd_v, [rows, cg + 1])
                t2 = t2 + mp * plsc.load_gather(dd_v, [rows, cg + 2])
                return acc, s0, s1, s2, t0, t1, t2

            acc, s0, s1, s2, t0, t1, t2 = lax.fori_loop(
                0, _DIM, dstep, (acc0, zero, zero, zero, zero, zero, zero),
                unroll=4)
            o_v[pl.ds(g * _L, _L)] = acc + s0 * t0 + s1 * t1 + s2 * t2
            return 0

        lax.fori_loop(0, _CH // _L, group, 0)
        pltpu.sync_copy(o_v, out.at[pl.ds(base + c * _CH, _CH)])


@jax.jit
def _run(ij0, ij1, ipx, m_bar, d_bar, m_tab, d_tab):
    m_pad, d_pack = pl.pallas_call(
        _prep_body,
        grid=(pl.cdiv(_ND, _PR),),
        in_specs=[
            pl.BlockSpec((_DIM, _PR), lambda i: (0, i)),
            pl.BlockSpec((_DW, _PR), lambda i: (0, i)),
        ],
        out_specs=[
            pl.BlockSpec((_PR, 128), lambda i: (i, 0)),
            pl.BlockSpec((_PR, 256), lambda i: (i, 0)),
        ],
        out_shape=[
            jax.ShapeDtypeStruct((_NM, 128), jnp.float32),
            jax.ShapeDtypeStruct((_ND, 256), jnp.float32),
        ],
    )(m_tab.T, d_tab.T)

    mesh = plsc.VectorSubcoreMesh(core_axis_name="c", subcore_axis_name="s")
    f = pl.kernel(
        _sc_body,
        out_type=jax.ShapeDtypeStruct((_B,), jnp.float32),
        mesh=mesh,
        scratch_types=[
            pltpu.VMEM((_RPW,), jnp.int32),
            pltpu.VMEM((_RPW,), jnp.int32),
            pltpu.VMEM((_RPW,), jnp.int32),
            pltpu.VMEM((_CH, 128), jnp.float32),
            pltpu.VMEM((_CH, 128), jnp.float32),
            pltpu.VMEM((_CH, 256), jnp.float32),
            pltpu.VMEM((_CH,), jnp.float32),
            pltpu.VMEM((_CH,), jnp.float32),
            pltpu.VMEM((_CH,), jnp.float32),
            pltpu.SemaphoreType.DMA,
        ],
        compiler_params=pltpu.CompilerParams(needs_layout_passes=False),
    )
    return f(ij0, ij1, ipx, m_bar, d_bar, m_pad, d_pack)


def kernel(ij, ip, m_bar, d_bar, M_table, D_table):
    ij0 = jnp.asarray(ij[:, 0], jnp.int32)
    ij1 = jnp.asarray(ij[:, 1], jnp.int32)
    return _run(ij0, ij1, ip, m_bar, d_bar, M_table, D_table)


# speedup vs baseline: 2.3690x; 1.0805x over previous
import jax
import jax.numpy as jnp
from jax import lax
from jax.experimental import pallas as pl
from jax.experimental.pallas import tpu as pltpu
from jax.experimental.pallas import tpu_sc as plsc

_B = 16384
_DIM = 32
_K = 3
_DW = _DIM * (2 * _K + 1)
_NM = 100000
_ND = 100000
_NC, _NS, _L = 2, 16, 16
_NW = _NC * _NS
_RPW = _B // _NW
_CH = 128
_NCH = _RPW // _CH
_PR = 4096


def _prep_body(mt_ref, dt_ref, mp_ref, dp_ref):
    m_t = jnp.transpose(mt_ref[...], (1, 0))
    mp_ref[...] = jnp.concatenate(
        [m_t, jnp.zeros((_PR, 128 - _DIM), jnp.float32)], axis=1)
    d_t = jnp.transpose(dt_ref[...], (1, 0))
    dp_ref[...] = jnp.concatenate(
        [d_t, jnp.zeros((_PR, 256 - _DW), jnp.float32)], axis=1)


def _sc_body(ij0, ij1, ipx, m_bar, d_bar, m_pack, d_pack, out,
             idx0_v, idx1_v, idxp_v, mi_v, mp_v, dd_v,
             mb_v, db_v, o_v, sem):
    wid = lax.axis_index("s") * _NC + lax.axis_index("c")
    base = wid * _RPW
    pltpu.sync_copy(ij0.at[pl.ds(base, _RPW)], idx0_v)
    pltpu.sync_copy(ij1.at[pl.ds(base, _RPW)], idx1_v)
    pltpu.sync_copy(ipx.at[pl.ds(base, _RPW)], idxp_v)
    iota = lax.broadcasted_iota(jnp.int32, (_L,), 0)

    for c in range(_NCH):
        i0 = idx0_v.at[pl.ds(c * _CH, _CH)]
        i1 = idx1_v.at[pl.ds(c * _CH, _CH)]
        ipc = idxp_v.at[pl.ds(c * _CH, _CH)]
        cps = [
            pltpu.async_copy(d_pack.at[i1], dd_v, sem),
            pltpu.async_copy(m_pack.at[i0], mi_v, sem),
            pltpu.async_copy(m_pack.at[ipc], mp_v, sem),
            pltpu.async_copy(m_bar.at[i0], mb_v, sem),
            pltpu.async_copy(d_bar.at[i1], db_v, sem),
        ]
        for cp in cps:
            cp.wait()

        def group(g, _):
            rows = g * _L + iota
            acc0 = mb_v[pl.ds(g * _L, _L)] + db_v[pl.ds(g * _L, _L)]
            zero = jnp.zeros((_L,), jnp.float32)

            def dstep(t, carry):
                acc, s0, s1, s2, t0, t1, t2 = carry
                dv = jnp.bitwise_and(t + iota, _DIM - 1)
                mi = plsc.load_gather(mi_v, [rows, dv])
                mp = plsc.load_gather(mp_v, [rows, dv])
                dj = plsc.load_gather(dd_v, [rows, dv])
                acc = acc + mi * dj
                cs = _DIM + dv + dv + dv
                s0 = s0 + mi * plsc.load_gather(dd_v, [rows, cs])
                s1 = s1 + mi * plsc.load_gather(dd_v, [rows, cs + 1])
                s2 = s2 + mi * plsc.load_gather(dd_v, [rows, cs + 2])
                cg = cs + _K * _DIM
                t0 = t0 + mp * plsc.load_gather(dd_v, [rows, cg])
                t1 = t1 + mp * plsc.load_gather(dd_v, [rows, cg + 1])
                t2 = t2 + mp * plsc.load_gather(dd_v, [rows, cg + 2])
                return acc, s0, s1, s2, t0, t1, t2

            acc, s0, s1, s2, t0, t1, t2 = lax.fori_loop(
                0, _DIM, dstep, (acc0, zero, zero, zero, zero, zero, zero),
                unroll=4)
            o_v[pl.ds(g * _L, _L)] = acc + s0 * t0 + s1 * t1 + s2 * t2
            return 0

        lax.fori_loop(0, _CH // _L, group, 0)
        pltpu.sync_copy(o_v, out.at[pl.ds(base + c * _CH, _CH)])


@jax.jit
def _run(ij0, ij1, ipx, m_bar, d_bar, m_tab, d_tab):
    m_pack, d_pack = pl.pallas_call(
        _prep_body,
        grid=(pl.cdiv(_ND, _PR),),
        in_specs=[
            pl.BlockSpec((_DIM, _PR), lambda i: (0, i)),
            pl.BlockSpec((_DW, _PR), lambda i: (0, i)),
        ],
        out_specs=[
            pl.BlockSpec((_PR, 128), lambda i: (i, 0)),
            pl.BlockSpec((_PR, 256), lambda i: (i, 0)),
        ],
        out_shape=[
            jax.ShapeDtypeStruct((_NM, 128), jnp.float32),
            jax.ShapeDtypeStruct((_ND, 256), jnp.float32),
        ],
    )(m_tab.T, d_tab.T)

    mesh = plsc.VectorSubcoreMesh(core_axis_name="c", subcore_axis_name="s")
    f = pl.kernel(
        _sc_body,
        out_type=jax.ShapeDtypeStruct((_B,), jnp.float32),
        mesh=mesh,
        scratch_types=[
            pltpu.VMEM((_RPW,), jnp.int32),
            pltpu.VMEM((_RPW,), jnp.int32),
            pltpu.VMEM((_RPW,), jnp.int32),
            pltpu.VMEM((_CH, 128), jnp.float32),
            pltpu.VMEM((_CH, 128), jnp.float32),
            pltpu.VMEM((_CH, 256), jnp.float32),
            pltpu.VMEM((_CH,), jnp.float32),
            pltpu.VMEM((_CH,), jnp.float32),
            pltpu.VMEM((_CH,), jnp.float32),
            pltpu.SemaphoreType.DMA,
        ],
        compiler_params=pltpu.CompilerParams(needs_layout_passes=False),
    )
    return f(ij0, ij1, ipx, m_bar, d_bar, m_pack, d_pack)


def kernel(ij, ip, m_bar, d_bar, M_table, D_table):
    ij0 = jnp.asarray(ij[:, 0], jnp.int32)
    ij1 = jnp.asarray(ij[:, 1], jnp.int32)
    return _run(ij0, ij1, ip, m_bar, d_bar, M_table, D_table)


# speedup vs baseline: 2.4093x; 1.0170x over previous
import jax
import jax.numpy as jnp
from jax import lax
from jax.experimental import pallas as pl
from jax.experimental.pallas import tpu as pltpu
from jax.experimental.pallas import tpu_sc as plsc

_B = 16384
_DIM = 32
_K = 3
_DW = _DIM * (2 * _K + 1)
_NM = 100000
_ND = 100000
_NC, _NS, _L = 2, 16, 16
_NW = _NC * _NS
_RPW = _B // _NW
_CH = 128
_NCH = _RPW // _CH
_PR = 8192


def _prep_body(mt_ref, dt_ref, mp_ref, dp_ref):
    m_t = jnp.transpose(mt_ref[...], (1, 0))
    mp_ref[...] = jnp.concatenate(
        [m_t, jnp.zeros((_PR, 128 - _DIM), jnp.float32)], axis=1)
    d_t = jnp.transpose(dt_ref[...], (1, 0))
    dp_ref[...] = jnp.concatenate(
        [d_t, jnp.zeros((_PR, 256 - _DW), jnp.float32)], axis=1)


def _sc_body(ij0, ij1, ipx, m_bar, d_bar, m_pack, d_pack, out,
             idx0_v, idx1_v, idxp_v, mi_v, mp_v, dd_v,
             mb_v, db_v, o_v, sem):
    wid = lax.axis_index("s") * _NC + lax.axis_index("c")
    base = wid * _RPW
    pltpu.sync_copy(ij0.at[pl.ds(base, _RPW)], idx0_v)
    pltpu.sync_copy(ij1.at[pl.ds(base, _RPW)], idx1_v)
    pltpu.sync_copy(ipx.at[pl.ds(base, _RPW)], idxp_v)
    iota = lax.broadcasted_iota(jnp.int32, (_L,), 0)

    for c in range(_NCH):
        i0 = idx0_v.at[pl.ds(c * _CH, _CH)]
        i1 = idx1_v.at[pl.ds(c * _CH, _CH)]
        ipc = idxp_v.at[pl.ds(c * _CH, _CH)]
        cps = [
            pltpu.async_copy(d_pack.at[i1], dd_v, sem),
            pltpu.async_copy(m_pack.at[i0], mi_v, sem),
            pltpu.async_copy(m_pack.at[ipc], mp_v, sem),
            pltpu.async_copy(m_bar.at[i0], mb_v, sem),
            pltpu.async_copy(d_bar.at[i1], db_v, sem),
        ]
        for cp in cps:
            cp.wait()

        def group(g, _):
            rows = g * _L + iota
            acc0 = mb_v[pl.ds(g * _L, _L)] + db_v[pl.ds(g * _L, _L)]
            zero = jnp.zeros((_L,), jnp.float32)

            def dstep(t, carry):
                acc, s0, s1, s2, t0, t1, t2 = carry
                dv = jnp.bitwise_and(t + iota, _DIM - 1)
                mi = plsc.load_gather(mi_v, [rows, dv])
                mp = plsc.load_gather(mp_v, [rows, dv])
                dj = plsc.load_gather(dd_v, [rows, dv])
                acc = acc + mi * dj
                cs = _DIM + dv + dv + dv
                s0 = s0 + mi * plsc.load_gather(dd_v, [rows, cs])
                s1 = s1 + mi * plsc.load_gather(dd_v, [rows, cs + 1])
                s2 = s2 + mi * plsc.load_gather(dd_v, [rows, cs + 2])
                cg = cs + _K * _DIM
                t0 = t0 + mp * plsc.load_gather(dd_v, [rows, cg])
                t1 = t1 + mp * plsc.load_gather(dd_v, [rows, cg + 1])
                t2 = t2 + mp * plsc.load_gather(dd_v, [rows, cg + 2])
                return acc, s0, s1, s2, t0, t1, t2

            acc, s0, s1, s2, t0, t1, t2 = lax.fori_loop(
                0, _DIM, dstep, (acc0, zero, zero, zero, zero, zero, zero),
                unroll=4)
            o_v[pl.ds(g * _L, _L)] = acc + s0 * t0 + s1 * t1 + s2 * t2
            return 0

        lax.fori_loop(0, _CH // _L, group, 0)
        pltpu.sync_copy(o_v, out.at[pl.ds(base + c * _CH, _CH)])


@jax.jit
def _run(ij0, ij1, ipx, m_bar, d_bar, m_tab, d_tab):
    m_pack, d_pack = pl.pallas_call(
        _prep_body,
        grid=(pl.cdiv(_ND, _PR),),
        in_specs=[
            pl.BlockSpec((_DIM, _PR), lambda i: (0, i)),
            pl.BlockSpec((_DW, _PR), lambda i: (0, i)),
        ],
        out_specs=[
            pl.BlockSpec((_PR, 128), lambda i: (i, 0)),
            pl.BlockSpec((_PR, 256), lambda i: (i, 0)),
        ],
        out_shape=[
            jax.ShapeDtypeStruct((_NM, 128), jnp.float32),
            jax.ShapeDtypeStruct((_ND, 256), jnp.float32),
        ],
    )(m_tab.T, d_tab.T)

    mesh = plsc.VectorSubcoreMesh(core_axis_name="c", subcore_axis_name="s")
    f = pl.kernel(
        _sc_body,
        out_type=jax.ShapeDtypeStruct((_B,), jnp.float32),
        mesh=mesh,
        scratch_types=[
            pltpu.VMEM((_RPW,), jnp.int32),
            pltpu.VMEM((_RPW,), jnp.int32),
            pltpu.VMEM((_RPW,), jnp.int32),
            pltpu.VMEM((_CH, 128), jnp.float32),
            pltpu.VMEM((_CH, 128), jnp.float32),
            pltpu.VMEM((_CH, 256), jnp.float32),
            pltpu.VMEM((_CH,), jnp.float32),
            pltpu.VMEM((_CH,), jnp.float32),
            pltpu.VMEM((_CH,), jnp.float32),
            pltpu.SemaphoreType.DMA,
        ],
        compiler_params=pltpu.CompilerParams(needs_layout_passes=False),
    )
    return f(ij0, ij1, ipx, m_bar, d_bar, m_pack, d_pack)


def kernel(ij, ip, m_bar, d_bar, M_table, D_table):
    ij0 = jnp.asarray(ij[:, 0], jnp.int32)
    ij1 = jnp.asarray(ij[:, 1], jnp.int32)
    return _run(ij0, ij1, ip, m_bar, d_bar, M_table, D_table)


# speedup vs baseline: 2.4860x; 1.0319x over previous
import jax
import jax.numpy as jnp
from jax import lax
from jax.experimental import pallas as pl
from jax.experimental.pallas import tpu as pltpu
from jax.experimental.pallas import tpu_sc as plsc

_B = 16384
_DIM = 32
_K = 3
_DW = _DIM * (2 * _K + 1)
_NM = 100000
_ND = 100000
_NC, _NS, _L = 2, 16, 16
_NW = _NC * _NS
_RPW = _B // _NW
_CH = 64
_NCH = _RPW // _CH
_PR = 8192


def _prep_body(mt_ref, dt_ref, mp_ref, dp_ref):
    m_t = jnp.transpose(mt_ref[...], (1, 0))
    mp_ref[...] = jnp.concatenate(
        [m_t, jnp.zeros((_PR, 128 - _DIM), jnp.float32)], axis=1)
    d_t = jnp.transpose(dt_ref[...], (1, 0))
    dp_ref[...] = jnp.concatenate(
        [d_t, jnp.zeros((_PR, 256 - _DW), jnp.float32)], axis=1)


def _sc_body(ij0, ij1, ipx, m_bar, d_bar, m_pack, d_pack, out,
             idx0_v, idx1_v, idxp_v,
             mi0, mp0, dd0, mb0, db0, o0,
             mi1, mp1, dd1, mb1, db1, o1,
             sem0, sem1, osem):
    wid = lax.axis_index("s") * _NC + lax.axis_index("c")
    base = wid * _RPW
    pltpu.sync_copy(ij0.at[pl.ds(base, _RPW)], idx0_v)
    pltpu.sync_copy(ij1.at[pl.ds(base, _RPW)], idx1_v)
    pltpu.sync_copy(ipx.at[pl.ds(base, _RPW)], idxp_v)
    iota = lax.broadcasted_iota(jnp.int32, (_L,), 0)
    bufs = [(mi0, mp0, dd0, mb0, db0, o0, sem0),
            (mi1, mp1, dd1, mb1, db1, o1, sem1)]

    def fire(c):
        i0 = idx0_v.at[pl.ds(c * _CH, _CH)]
        i1 = idx1_v.at[pl.ds(c * _CH, _CH)]
        ipc = idxp_v.at[pl.ds(c * _CH, _CH)]
        mi_v, mp_v, dd_v, mb_v, db_v, _, sem = bufs[c % 2]
        return [
            pltpu.async_copy(d_pack.at[i1], dd_v, sem),
            pltpu.async_copy(m_pack.at[i0], mi_v, sem),
            pltpu.async_copy(m_pack.at[ipc], mp_v, sem),
            pltpu.async_copy(m_bar.at[i0], mb_v, sem),
            pltpu.async_copy(d_bar.at[i1], db_v, sem),
        ]

    pend = fire(0)
    ocp = [None, None]
    for c in range(_NCH):
        mi_v, mp_v, dd_v, mb_v, db_v, o_v, _ = bufs[c % 2]
        nxt = fire(c + 1) if c + 1 < _NCH else None
        for cp in pend:
            cp.wait()
        if ocp[c % 2] is not None:
            ocp[c % 2].wait()

        def group(g, _):
            rows = g * _L + iota
            acc0 = mb_v[pl.ds(g * _L, _L)] + db_v[pl.ds(g * _L, _L)]
            zero = jnp.zeros((_L,), jnp.float32)

            def dstep(t, carry):
                acc, s0, s1, s2, t0, t1, t2 = carry
                dv = jnp.bitwise_and(t + iota, _DIM - 1)
                mi = plsc.load_gather(mi_v, [rows, dv])
                mp = plsc.load_gather(mp_v, [rows, dv])
                dj = plsc.load_gather(dd_v, [rows, dv])
                acc = acc + mi * dj
                cs = _DIM + dv + dv + dv
                s0 = s0 + mi * plsc.load_gather(dd_v, [rows, cs])
                s1 = s1 + mi * plsc.load_gather(dd_v, [rows, cs + 1])
                s2 = s2 + mi * plsc.load_gather(dd_v, [rows, cs + 2])
                cg = cs + _K * _DIM
                t0 = t0 + mp * plsc.load_gather(dd_v, [rows, cg])
                t1 = t1 + mp * plsc.load_gather(dd_v, [rows, cg + 1])
                t2 = t2 + mp * plsc.load_gather(dd_v, [rows, cg + 2])
                return acc, s0, s1, s2, t0, t1, t2

            acc, s0, s1, s2, t0, t1, t2 = lax.fori_loop(
                0, _DIM, dstep, (acc0, zero, zero, zero, zero, zero, zero),
                unroll=4)
            o_v[pl.ds(g * _L, _L)] = acc + s0 * t0 + s1 * t1 + s2 * t2
            return 0

        lax.fori_loop(0, _CH // _L, group, 0)
        ocp[c % 2] = pltpu.async_copy(
            o_v, out.at[pl.ds(base + c * _CH, _CH)], osem)
        pend = nxt
    for cp in ocp:
        if cp is not None:
            cp.wait()


@jax.jit
def _run(ij0, ij1, ipx, m_bar, d_bar, m_tab, d_tab):
    m_pack, d_pack = pl.pallas_call(
        _prep_body,
        grid=(pl.cdiv(_ND, _PR),),
        in_specs=[
            pl.BlockSpec((_DIM, _PR), lambda i: (0, i)),
            pl.BlockSpec((_DW, _PR), lambda i: (0, i)),
        ],
        out_specs=[
            pl.BlockSpec((_PR, 128), lambda i: (i, 0)),
            pl.BlockSpec((_PR, 256), lambda i: (i, 0)),
        ],
        out_shape=[
            jax.ShapeDtypeStruct((_NM, 128), jnp.float32),
            jax.ShapeDtypeStruct((_ND, 256), jnp.float32),
        ],
    )(m_tab.T, d_tab.T)

    mesh = plsc.VectorSubcoreMesh(core_axis_name="c", subcore_axis_name="s")
    f = pl.kernel(
        _sc_body,
        out_type=jax.ShapeDtypeStruct((_B,), jnp.float32),
        mesh=mesh,
        scratch_types=[
            pltpu.VMEM((_RPW,), jnp.int32),
            pltpu.VMEM((_RPW,), jnp.int32),
            pltpu.VMEM((_RPW,), jnp.int32),
            pltpu.VMEM((_CH, 128), jnp.float32),
            pltpu.VMEM((_CH, 128), jnp.float32),
            pltpu.VMEM((_CH, 256), jnp.float32),
            pltpu.VMEM((_CH,), jnp.float32),
            pltpu.VMEM((_CH,), jnp.float32),
            pltpu.VMEM((_CH,), jnp.float32),
            pltpu.VMEM((_CH, 128), jnp.float32),
            pltpu.VMEM((_CH, 128), jnp.float32),
            pltpu.VMEM((_CH, 256), jnp.float32),
            pltpu.VMEM((_CH,), jnp.float32),
            pltpu.VMEM((_CH,), jnp.float32),
            pltpu.VMEM((_CH,), jnp.float32),
            pltpu.SemaphoreType.DMA,
            pltpu.SemaphoreType.DMA,
            pltpu.SemaphoreType.DMA,
        ],
        compiler_params=pltpu.CompilerParams(needs_layout_passes=False),
    )
    return f(ij0, ij1, ipx, m_bar, d_bar, m_pack, d_pack)


def kernel(ij, ip, m_bar, d_bar, M_table, D_table):
    ij0 = jnp.asarray(ij[:, 0], jnp.int32)
    ij1 = jnp.asarray(ij[:, 1], jnp.int32)
    return _run(ij0, ij1, ip, m_bar, d_bar, M_table, D_table)
